# baseline (speedup 1.0000x reference)
import functools

import jax
import jax.numpy as jnp
from jax import lax
from jax.experimental import pallas as pl
from jax.experimental.pallas import tpu as pltpu
from jax.experimental.pallas import tpu_sc as plsc

B, C, H, W = 8, 384, 224, 224
HID, E, K = 96, 64, 2
HWTOT = H * W
CSL = 16
BH = 2
BPH = B // BH
NCS = C // CSL
HS = 2
HSL = H // HS
NSLAB = BH * NCS * HS
NBUF = 6


LBLK = 512
NSTEP = HWTOT // LBLK


def _pool_body(x_ref, out_ref, acc_ref):
    j = pl.program_id(0)

    @pl.when(j == 0)
    def _init():
        acc_ref[...] = jnp.zeros_like(acc_ref)

    acc_ref[...] += jnp.sum(x_ref[...], axis=1, keepdims=True)

    @pl.when(j == NSTEP - 1)
    def _done():
        out_ref[...] = acc_ref[...]


def _head_body(h_ref, w1_ref, b1_ref, w2_ref, b2_ref, logits_ref):
    h = h_ref[...] * (1.0 / HWTOT)
    hid = jax.lax.dot_general(h, w1_ref[...],
                              (((1,), (1,)), ((), ())),
                              preferred_element_type=jnp.float32)
    hid = jnp.maximum(hid + b1_ref[...], 0.0)
    logits = jax.lax.dot_general(hid, w2_ref[...],
                                 (((1,), (1,)), ((), ())),
                                 preferred_element_type=jnp.float32)
    logits_ref[...] = logits + b2_ref[...]


EC = E // 16


def _sc_router_body(logits_hbm, idx_hbm, val_hbm, probs_hbm,
                    row_v, probs_v, val16_v, idx16_v):
    wid = lax.axis_index("s") * 2 + lax.axis_index("c")

    @pl.when(wid < B)
    def _route_row():
        b = wid
        pltpu.sync_copy(logits_hbm.at[pl.ds(b * E, E)], row_v)
        chunks = [row_v[pl.ds(16 * k, 16)] for k in range(EC)]
        iotas = [lax.iota(jnp.int32, 16) + 16 * k for k in range(EC)]

        m = chunks[0]
        for ck in chunks[1:]:
            m = jnp.maximum(m, ck)
        m1 = jnp.max(m)
        es = [jnp.exp(ck - m1) for ck in chunks]
        tot = es[0]
        for ek in es[1:]:
            tot = tot + ek
        denom = jnp.sum(tot)
        ps = [ek / denom for ek in es]
        for k in range(EC):
            probs_v[pl.ds(16 * k, 16)] = ps[k]

        pm = ps[0]
        for pk in ps[1:]:
            pm = jnp.maximum(pm, pk)
        pm1 = jnp.max(pm)
        cand = [jnp.where(ps[k] == pm1, iotas[k], E) for k in range(EC)]
        cm = cand[0]
        for c2 in cand[1:]:
            cm = jnp.minimum(cm, c2)
        i1 = jnp.min(cm)

        neg = jnp.float32(-jnp.inf)
        ms = [jnp.where(iotas[k] == i1, neg, ps[k]) for k in range(EC)]
        mm = ms[0]
        for mk in ms[1:]:
            mm = jnp.maximum(mm, mk)
        pm2 = jnp.max(mm)
        cand2 = [jnp.where(ms[k] == pm2, iotas[k], E) for k in range(EC)]
        cm2 = cand2[0]
        for c2 in cand2[1:]:
            cm2 = jnp.minimum(cm2, c2)
        i2 = jnp.min(cm2)

        i16 = lax.iota(jnp.int32, 16)
        val16_v[...] = jnp.where(i16 == 0, pm1,
                                 jnp.where(i16 == 1, pm2, 0.0))
        idx16_v[...] = jnp.where(i16 == 0, i1,
                                 jnp.where(i16 == 1, i2, 0))
        pltpu.sync_copy(probs_v, probs_hbm.at[pl.ds(b * E, E)])
        pltpu.sync_copy(val16_v.at[pl.ds(0, 8)], val_hbm.at[pl.ds(b * 8, 8)])
        pltpu.sync_copy(idx16_v.at[pl.ds(0, 8)], idx_hbm.at[pl.ds(b * 8, 8)])


_sc_router = functools.partial(
    pl.kernel,
    out_type=[jax.ShapeDtypeStruct((B * 8,), jnp.int32),
              jax.ShapeDtypeStruct((B * 8,), jnp.float32),
              jax.ShapeDtypeStruct((B * E,), jnp.float32)],
    mesh=plsc.VectorSubcoreMesh(core_axis_name="c", subcore_axis_name="s",
                                num_cores=2, num_subcores=16),
    scratch_types=[pltpu.VMEM((E,), jnp.float32),
                   pltpu.VMEM((E,), jnp.float32),
                   pltpu.VMEM((16,), jnp.float32),
                   pltpu.VMEM((16,), jnp.int32)],
    compiler_params=pltpu.CompilerParams(needs_layout_passes=False),
)(_sc_router_body)


@jax.jit
def kernel(x, W1, b1, W2, b2):
    x2 = x.reshape(B * C, HWTOT)
    sums = pl.pallas_call(
        _pool_body,
        grid=(NSTEP,),
        in_specs=[pl.BlockSpec((B * C, LBLK), lambda j: (0, j))],
        out_specs=pl.BlockSpec((B * C, 1), lambda j: (0, 0)),
        out_shape=jax.ShapeDtypeStruct((B * C, 1), jnp.float32),
        scratch_shapes=[pltpu.VMEM((B * C, 1), jnp.float32)],
    )(x2)

    h = sums.reshape(B, C)

    logits = pl.pallas_call(
        _head_body,
        in_specs=[pl.BlockSpec((B, C), lambda: (0, 0)),
                  pl.BlockSpec(W1.shape, lambda: (0, 0)),
                  pl.BlockSpec((1, HID), lambda: (0, 0)),
                  pl.BlockSpec(W2.shape, lambda: (0, 0)),
                  pl.BlockSpec((1, E), lambda: (0, 0))],
        out_specs=pl.BlockSpec((B, E), lambda: (0, 0)),
        out_shape=jax.ShapeDtypeStruct((B, E), jnp.float32),
    )(h, W1, b1.reshape(1, HID), W2, b2.reshape(1, E))

    idx1, val1, probs1 = _sc_router(logits.reshape(B * E))
    return (idx1.reshape(B, 8)[:, :K], val1.reshape(B, 8)[:, :K],
            probs1.reshape(B, E))


# speedup vs baseline: 1.8391x; 1.8391x over previous
import functools

import jax
import jax.numpy as jnp
from jax import lax
from jax.experimental import pallas as pl
from jax.experimental.pallas import tpu as pltpu
from jax.experimental.pallas import tpu_sc as plsc

B, C, H, W = 8, 384, 224, 224
HID, E, K = 96, 64, 2
HWTOT = H * W
CSL = 16
BH = 2
BPH = B // BH
NCS = C // CSL
HS = 2
HSL = H // HS
NSLAB = BH * NCS * HS
NBUF = 6


LBLK = 512
NSTEP = HWTOT // LBLK


def _pool_body(x_ref, out_ref, acc_ref):
    j = pl.program_id(0)

    @pl.when(j == 0)
    def _init():
        acc_ref[...] = jnp.zeros_like(acc_ref)

    acc_ref[...] += jnp.sum(x_ref[...], axis=2)

    @pl.when(j == NSTEP - 1)
    def _done():
        out_ref[...] = acc_ref[...]


def _head_body(h_ref, w1_ref, b1_ref, w2_ref, b2_ref, logits_ref):
    h = h_ref[...] * (1.0 / HWTOT)
    hid = jax.lax.dot_general(h, w1_ref[...],
                              (((1,), (1,)), ((), ())),
                              preferred_element_type=jnp.float32)
    hid = jnp.maximum(hid + b1_ref[...], 0.0)
    logits = jax.lax.dot_general(hid, w2_ref[...],
                                 (((1,), (1,)), ((), ())),
                                 preferred_element_type=jnp.float32)
    logits_ref[...] = logits + b2_ref[...]


EC = E // 16


def _sc_router_body(logits_hbm, idx_hbm, val_hbm, probs_hbm,
                    row_v, probs_v, val16_v, idx16_v):
    wid = lax.axis_index("s") * 2 + lax.axis_index("c")

    @pl.when(wid < B)
    def _route_row():
        b = wid
        pltpu.sync_copy(logits_hbm.at[pl.ds(b * E, E)], row_v)
        chunks = [row_v[pl.ds(16 * k, 16)] for k in range(EC)]
        iotas = [lax.iota(jnp.int32, 16) + 16 * k for k in range(EC)]

        m = chunks[0]
        for ck in chunks[1:]:
            m = jnp.maximum(m, ck)
        m1 = jnp.max(m)
        es = [jnp.exp(ck - m1) for ck in chunks]
        tot = es[0]
        for ek in es[1:]:
            tot = tot + ek
        denom = jnp.sum(tot)
        ps = [ek / denom for ek in es]
        for k in range(EC):
            probs_v[pl.ds(16 * k, 16)] = ps[k]

        pm = ps[0]
        for pk in ps[1:]:
            pm = jnp.maximum(pm, pk)
        pm1 = jnp.max(pm)
        cand = [jnp.where(ps[k] == pm1, iotas[k], E) for k in range(EC)]
        cm = cand[0]
        for c2 in cand[1:]:
            cm = jnp.minimum(cm, c2)
        i1 = jnp.min(cm)

        neg = jnp.float32(-jnp.inf)
        ms = [jnp.where(iotas[k] == i1, neg, ps[k]) for k in range(EC)]
        mm = ms[0]
        for mk in ms[1:]:
            mm = jnp.maximum(mm, mk)
        pm2 = jnp.max(mm)
        cand2 = [jnp.where(ms[k] == pm2, iotas[k], E) for k in range(EC)]
        cm2 = cand2[0]
        for c2 in cand2[1:]:
            cm2 = jnp.minimum(cm2, c2)
        i2 = jnp.min(cm2)

        i16 = lax.iota(jnp.int32, 16)
        val16_v[...] = jnp.where(i16 == 0, pm1,
                                 jnp.where(i16 == 1, pm2, 0.0))
        idx16_v[...] = jnp.where(i16 == 0, i1,
                                 jnp.where(i16 == 1, i2, 0))
        pltpu.sync_copy(probs_v, probs_hbm.at[pl.ds(b * E, E)])
        pltpu.sync_copy(val16_v.at[pl.ds(0, 8)], val_hbm.at[pl.ds(b * 8, 8)])
        pltpu.sync_copy(idx16_v.at[pl.ds(0, 8)], idx_hbm.at[pl.ds(b * 8, 8)])


_sc_router = functools.partial(
    pl.kernel,
    out_type=[jax.ShapeDtypeStruct((B * 8,), jnp.int32),
              jax.ShapeDtypeStruct((B * 8,), jnp.float32),
              jax.ShapeDtypeStruct((B * E,), jnp.float32)],
    mesh=plsc.VectorSubcoreMesh(core_axis_name="c", subcore_axis_name="s",
                                num_cores=2, num_subcores=16),
    scratch_types=[pltpu.VMEM((E,), jnp.float32),
                   pltpu.VMEM((E,), jnp.float32),
                   pltpu.VMEM((16,), jnp.float32),
                   pltpu.VMEM((16,), jnp.int32)],
    compiler_params=pltpu.CompilerParams(needs_layout_passes=False),
)(_sc_router_body)


@jax.jit
def kernel(x, W1, b1, W2, b2):
    x3 = x.reshape(B, C, HWTOT)
    sums = pl.pallas_call(
        _pool_body,
        grid=(NSTEP,),
        in_specs=[pl.BlockSpec((B, C, LBLK), lambda j: (0, 0, j))],
        out_specs=pl.BlockSpec((B, C), lambda j: (0, 0)),
        out_shape=jax.ShapeDtypeStruct((B, C), jnp.float32),
        scratch_shapes=[pltpu.VMEM((B, C), jnp.float32)],
    )(x3)

    h = sums

    logits = pl.pallas_call(
        _head_body,
        in_specs=[pl.BlockSpec((B, C), lambda: (0, 0)),
                  pl.BlockSpec(W1.shape, lambda: (0, 0)),
                  pl.BlockSpec((1, HID), lambda: (0, 0)),
                  pl.BlockSpec(W2.shape, lambda: (0, 0)),
                  pl.BlockSpec((1, E), lambda: (0, 0))],
        out_specs=pl.BlockSpec((B, E), lambda: (0, 0)),
        out_shape=jax.ShapeDtypeStruct((B, E), jnp.float32),
    )(h, W1, b1.reshape(1, HID), W2, b2.reshape(1, E))

    idx1, val1, probs1 = _sc_router(logits.reshape(B * E))
    return (idx1.reshape(B, 8)[:, :K], val1.reshape(B, 8)[:, :K],
            probs1.reshape(B, E))
